# Initial kernel scaffold; baseline (speedup 1.0000x reference)
import jax
import jax.numpy as jnp
from jax.experimental import pallas as pl


def kernel(x):
    raise NotImplementedError("write your pallas kernel here")


# speedup vs baseline: 1.0035x; 1.0035x over previous
import jax
import jax.numpy as jnp
from jax.experimental import pallas as pl
from jax.experimental.pallas import tpu as pltpu

_MAX_LEN = 200
_OBS = 128


def _body(x_ref, buf_ref, mask_ref):
    row = jax.lax.broadcasted_iota(jnp.int32, buf_ref.shape, 1)
    buf_ref[...] = jnp.where(row == 0, x_ref[...][:, None, :], 0.0)
    pos = jax.lax.broadcasted_iota(jnp.int32, mask_ref.shape, 1)
    mask_ref[...] = (pos >= _MAX_LEN - 1).astype(jnp.int32)


def kernel(x):
    batch = x.shape[0]
    bblk = 64
    grid = (batch // bblk,)
    buf, mask = pl.pallas_call(
        _body,
        grid=grid,
        in_specs=[pl.BlockSpec((bblk, _OBS), lambda i: (i, 0))],
        out_specs=[
            pl.BlockSpec((bblk, _MAX_LEN, _OBS), lambda i: (i, 0, 0)),
            pl.BlockSpec((1, _MAX_LEN), lambda i: (0, 0)),
        ],
        out_shape=[
            jax.ShapeDtypeStruct((batch, _MAX_LEN, _OBS), x.dtype),
            jax.ShapeDtypeStruct((1, _MAX_LEN), jnp.int32),
        ],
        compiler_params=pltpu.CompilerParams(
            dimension_semantics=("parallel",),
        ),
    )(x)
    return buf, (mask[0] != 0)
